# baseline (speedup 1.0000x reference)
import functools

import jax
import jax.numpy as jnp
from jax import lax
from jax.experimental import pallas as pl
from jax.experimental.pallas import tpu as pltpu
from jax.experimental.pallas import tpu_sc as plsc

N = 10000
E = 320000
NPAD = 10240
B = 112
EPAD = 322560
ROWS = EPAD // B
NC, NS = 2, 16
NW = NC * NS
CPT = ROWS // NW
NBUF = 5
SLICE = NPAD // NS
BN = 10240
GRID = NPAD // BN


def _sc_mesh():
    return plsc.VectorSubcoreMesh(core_axis_name="c", subcore_axis_name="s")


def _deg_call(e4, zeros1):
    @functools.partial(
        pl.kernel,
        out_type=jax.ShapeDtypeStruct((NC, NPAD), jnp.float32),
        mesh=_sc_mesh(),
        scratch_types=[
            pltpu.VMEM((CPT, B), jnp.int32),
            pltpu.VMEM((B,), jnp.float32),
            pltpu.VMEM_SHARED((NPAD,), jnp.float32),
        ],
    )
    def deg_kernel(e4_hbm, zeros_hbm, out_hbm, idx_v, ones_v, acc):
        c = lax.axis_index("c")
        sid = lax.axis_index("s")
        w = c * NS + sid
        pltpu.sync_copy(e4_hbm.at[1, w], idx_v)
        for j in range(B // 16):
            ones_v[pl.ds(j * 16, 16)] = jnp.ones((16,), jnp.float32)
        pltpu.sync_copy(zeros_hbm.at[pl.ds(sid * SLICE, SLICE)],
                        acc.at[pl.ds(sid * SLICE, SLICE)])
        plsc.subcore_barrier()

        def body(k, carry):
            pltpu.sync_copy(ones_v, acc.at[idx_v.at[k]], add=True)
            return carry

        lax.fori_loop(0, CPT, body, 0)
        plsc.subcore_barrier()
        pltpu.sync_copy(acc.at[pl.ds(sid * SLICE, SLICE)],
                        out_hbm.at[c, pl.ds(sid * SLICE, SLICE)])

    return deg_kernel(e4, zeros1)


def _agg_call(hp, e4, zeros2, F):
    @functools.partial(
        pl.kernel,
        out_type=jax.ShapeDtypeStruct((NC, NPAD, F), jnp.float32),
        mesh=_sc_mesh(),
        scratch_types=[
            pltpu.VMEM((CPT, B), jnp.int32),
            pltpu.VMEM((CPT, B), jnp.int32),
            pltpu.VMEM((NBUF, B, F), jnp.float32),
            pltpu.VMEM_SHARED((NPAD, F), jnp.float32),
        ] + [pltpu.SemaphoreType.DMA] * NBUF,
        compiler_params=pltpu.CompilerParams(use_tc_tiling_on_sc=False),
    )
    def agg_kernel(hp_hbm, e4_hbm, zeros_hbm, out_hbm,
                   srcv, dstv, rows_v, acc, s0, s1, s2, s3, s4):
        sems = (s0, s1, s2, s3, s4)
        c = lax.axis_index("c")
        sid = lax.axis_index("s")
        w = c * NS + sid
        pltpu.sync_copy(e4_hbm.at[0, w], srcv)
        pltpu.sync_copy(e4_hbm.at[1, w], dstv)
        pltpu.sync_copy(zeros_hbm.at[pl.ds(sid * SLICE, SLICE)],
                        acc.at[pl.ds(sid * SLICE, SLICE)])
        plsc.subcore_barrier()

        for b in range(NBUF):
            pltpu.async_copy(hp_hbm.at[srcv.at[b]], rows_v.at[b], sems[b])

        def group(g, carry):
            for b in range(NBUF):
                k = g * NBUF + b
                pltpu.make_async_copy(
                    hp_hbm.at[srcv.at[k]], rows_v.at[b], sems[b]).wait()
                pltpu.sync_copy(rows_v.at[b], acc.at[dstv.at[k]], add=True)
                pltpu.async_copy(
                    hp_hbm.at[srcv.at[k + NBUF]], rows_v.at[b], sems[b])
            return carry

        lax.fori_loop(0, CPT // NBUF - 1, group, 0)
        for b in range(NBUF):
            k = CPT - NBUF + b
            pltpu.make_async_copy(
                hp_hbm.at[srcv.at[k]], rows_v.at[b], sems[b]).wait()
            pltpu.sync_copy(rows_v.at[b], acc.at[dstv.at[k]], add=True)
        plsc.subcore_barrier()
        pltpu.sync_copy(acc.at[pl.ds(sid * SLICE, SLICE)],
                        out_hbm.at[c, pl.ds(sid * SLICE, SLICE)])

    return agg_kernel(hp, e4, zeros2)


def _tc1_body(x_ref, w_ref, deg_ref, hp_ref, dinv_ref):
    deg = deg_ref[0:1, :] + deg_ref[1:2, :] + 1.0
    dv = jnp.transpose(lax.rsqrt(deg))
    h = jnp.dot(x_ref[...], w_ref[...], preferred_element_type=jnp.float32)
    hp_ref[...] = h * dv
    dinv_ref[...] = dv


def _tc1_call(x, W1, deg2, Fin, F):
    return pl.pallas_call(
        _tc1_body,
        grid=(GRID,),
        in_specs=[
            pl.BlockSpec((BN, Fin), lambda i: (i, 0)),
            pl.BlockSpec((Fin, F), lambda i: (0, 0)),
            pl.BlockSpec((NC, BN), lambda i: (0, i)),
        ],
        out_specs=[
            pl.BlockSpec((BN, F), lambda i: (i, 0)),
            pl.BlockSpec((BN, 1), lambda i: (i, 0)),
        ],
        out_shape=[
            jax.ShapeDtypeStruct((NPAD, F), jnp.float32),
            jax.ShapeDtypeStruct((NPAD, 1), jnp.float32),
        ],
    )(x, W1, deg2)


def _tc_mid_body(agg_ref, hprev_ref, dinv_ref, w_ref, b_ref, hp_ref):
    dv = dinv_ref[...]
    a = agg_ref[0] + agg_ref[1] + hprev_ref[...]
    xl = jnp.maximum(dv * a + b_ref[...], 0.0)
    h = jnp.dot(xl, w_ref[...], preferred_element_type=jnp.float32)
    hp_ref[...] = h * dv


def _tc_mid_call(aggp, hprev, dinv, W, b, Fp, F):
    return pl.pallas_call(
        _tc_mid_body,
        grid=(GRID,),
        in_specs=[
            pl.BlockSpec((NC, BN, Fp), lambda i: (0, i, 0)),
            pl.BlockSpec((BN, Fp), lambda i: (i, 0)),
            pl.BlockSpec((BN, 1), lambda i: (i, 0)),
            pl.BlockSpec((Fp, F), lambda i: (0, 0)),
            pl.BlockSpec((1, Fp), lambda i: (0, 0)),
        ],
        out_specs=pl.BlockSpec((BN, F), lambda i: (i, 0)),
        out_shape=jax.ShapeDtypeStruct((NPAD, F), jnp.float32),
    )(aggp, hprev, dinv, W, b)


def _tc_fin_body(agg_ref, h4_ref, dinv_ref, b_ref, out_ref):
    dv = dinv_ref[...]
    a = agg_ref[0] + agg_ref[1] + h4_ref[...]
    o = dv * a + b_ref[...]
    v = o[:, 0:3]
    m = jnp.max(v, axis=1, keepdims=True)
    s = jnp.sum(jnp.exp(v - m), axis=1, keepdims=True)
    out_ref[...] = v - m - jnp.log(s)


def _tc_fin_call(aggp, h4, dinv, b4p, Fp):
    return pl.pallas_call(
        _tc_fin_body,
        grid=(GRID,),
        in_specs=[
            pl.BlockSpec((NC, BN, Fp), lambda i: (0, i, 0)),
            pl.BlockSpec((BN, Fp), lambda i: (i, 0)),
            pl.BlockSpec((BN, 1), lambda i: (i, 0)),
            pl.BlockSpec((1, Fp), lambda i: (0, 0)),
        ],
        out_specs=pl.BlockSpec((BN, 3), lambda i: (i, 0)),
        out_shape=jax.ShapeDtypeStruct((NPAD, 3), jnp.float32),
    )(aggp, h4, dinv, b4p)


def kernel(x, edge_index, W1, b1, W2, b2, W3, b3, W4, b4):
    f32 = jnp.float32
    pad_src = jnp.zeros((1, EPAD - E), jnp.int32)
    pad_dst = N + jax.lax.rem(jnp.arange(EPAD - E, dtype=jnp.int32),
                              jnp.int32(NPAD - N))[None]
    e4 = jnp.concatenate(
        [edge_index, jnp.concatenate([pad_src, pad_dst], axis=0)],
        axis=1).reshape(2, NW, CPT, B)
    xp = jnp.pad(x, ((0, NPAD - N), (0, 0)))

    W4p = jnp.zeros((64, 16), f32).at[:, 0:3].set(W4)
    b4p = jnp.zeros((1, 16), f32).at[0, 0:3].set(b4)

    zeros1 = jnp.zeros((NPAD,), f32)
    zf = {F: jnp.zeros((NPAD, F), f32) for F in (16, 32, 64)}

    deg2 = _deg_call(e4, zeros1)

    h1p, dinv = _tc1_call(xp, W1, deg2, 128, 16)
    a1 = _agg_call(h1p, e4, zf[16], 16)
    h2p = _tc_mid_call(a1, h1p, dinv, W2, b1.reshape(1, 16), 16, 32)
    a2 = _agg_call(h2p, e4, zf[32], 32)
    h3p = _tc_mid_call(a2, h2p, dinv, W3, b2.reshape(1, 32), 32, 64)
    a3 = _agg_call(h3p, e4, zf[64], 64)
    h4p = _tc_mid_call(a3, h3p, dinv, W4p, b3.reshape(1, 64), 64, 16)
    a4 = _agg_call(h4p, e4, zf[16], 16)
    out = _tc_fin_call(a4, h4p, dinv, b4p, 16)

    return out[:N]


# speedup vs baseline: 1.3080x; 1.3080x over previous
import functools

import jax
import jax.numpy as jnp
from jax import lax
from jax.experimental import pallas as pl
from jax.experimental.pallas import tpu as pltpu
from jax.experimental.pallas import tpu_sc as plsc

N = 10000
E = 320000
NPAD = 10240
B = 80
ROWS = E // B
NC, NS = 2, 16
NW = NC * NS
CPT = ROWS // NW
NBUF = 5
SLICE = NPAD // NS
BN = 10240
GRID = NPAD // BN


def _sc_mesh():
    return plsc.VectorSubcoreMesh(core_axis_name="c", subcore_axis_name="s")


def _deg_call(e4, zeros1):
    @functools.partial(
        pl.kernel,
        out_type=jax.ShapeDtypeStruct((NC, NPAD), jnp.float32),
        mesh=_sc_mesh(),
        scratch_types=[
            pltpu.VMEM((CPT, B), jnp.int32),
            pltpu.VMEM((B,), jnp.float32),
            pltpu.VMEM_SHARED((NPAD,), jnp.float32),
        ],
    )
    def deg_kernel(e4_hbm, zeros_hbm, out_hbm, idx_v, ones_v, acc):
        c = lax.axis_index("c")
        sid = lax.axis_index("s")
        w = c * NS + sid
        pltpu.sync_copy(e4_hbm.at[1, w], idx_v)
        for j in range(B // 16):
            ones_v[pl.ds(j * 16, 16)] = jnp.ones((16,), jnp.float32)
        pltpu.sync_copy(zeros_hbm.at[pl.ds(sid * SLICE, SLICE)],
                        acc.at[pl.ds(sid * SLICE, SLICE)])
        plsc.subcore_barrier()

        def body(k, carry):
            pltpu.sync_copy(ones_v, acc.at[idx_v.at[k]], add=True)
            return carry

        lax.fori_loop(0, CPT, body, 0)
        plsc.subcore_barrier()
        pltpu.sync_copy(acc.at[pl.ds(sid * SLICE, SLICE)],
                        out_hbm.at[c, pl.ds(sid * SLICE, SLICE)])

    return deg_kernel(e4, zeros1)


def _agg_call(hp, e4, zeros2, F):
    @functools.partial(
        pl.kernel,
        out_type=jax.ShapeDtypeStruct((NC, NPAD, F), jnp.float32),
        mesh=_sc_mesh(),
        scratch_types=[
            pltpu.VMEM((CPT, B), jnp.int32),
            pltpu.VMEM((CPT, B), jnp.int32),
            pltpu.VMEM((NBUF, B, F), jnp.float32),
            pltpu.VMEM_SHARED((NPAD, F), jnp.float32),
        ] + [pltpu.SemaphoreType.DMA] * NBUF,
        compiler_params=pltpu.CompilerParams(use_tc_tiling_on_sc=False),
    )
    def agg_kernel(hp_hbm, e4_hbm, zeros_hbm, out_hbm,
                   srcv, dstv, rows_v, acc, s0, s1, s2, s3, s4):
        sems = (s0, s1, s2, s3, s4)
        c = lax.axis_index("c")
        sid = lax.axis_index("s")
        w = c * NS + sid
        pltpu.sync_copy(e4_hbm.at[0, w], srcv)
        pltpu.sync_copy(e4_hbm.at[1, w], dstv)
        pltpu.sync_copy(zeros_hbm.at[pl.ds(sid * SLICE, SLICE)],
                        acc.at[pl.ds(sid * SLICE, SLICE)])
        plsc.subcore_barrier()

        for b in range(NBUF):
            pltpu.async_copy(hp_hbm.at[srcv.at[b]], rows_v.at[b], sems[b])

        def group(g, carry):
            for b in range(NBUF):
                k = g * NBUF + b
                pltpu.make_async_copy(
                    hp_hbm.at[srcv.at[k]], rows_v.at[b], sems[b]).wait()
                pltpu.sync_copy(rows_v.at[b], acc.at[dstv.at[k]], add=True)
                pltpu.async_copy(
                    hp_hbm.at[srcv.at[k + NBUF]], rows_v.at[b], sems[b])
            return carry

        lax.fori_loop(0, CPT // NBUF - 1, group, 0)
        for b in range(NBUF):
            k = CPT - NBUF + b
            pltpu.make_async_copy(
                hp_hbm.at[srcv.at[k]], rows_v.at[b], sems[b]).wait()
            pltpu.sync_copy(rows_v.at[b], acc.at[dstv.at[k]], add=True)
        plsc.subcore_barrier()
        pltpu.sync_copy(acc.at[pl.ds(sid * SLICE, SLICE)],
                        out_hbm.at[c, pl.ds(sid * SLICE, SLICE)])

    return agg_kernel(hp, e4, zeros2)


def _tc1_body(x_ref, w_ref, deg_ref, hp_ref, dinv_ref):
    deg = deg_ref[0:1, :] + deg_ref[1:2, :] + 1.0
    dv = jnp.transpose(lax.rsqrt(deg))
    h = jnp.dot(x_ref[...], w_ref[...], preferred_element_type=jnp.float32)
    hp_ref[...] = h * dv
    dinv_ref[...] = dv


def _tc1_call(x, W1, deg2, Fin, F):
    return pl.pallas_call(
        _tc1_body,
        grid=(GRID,),
        in_specs=[
            pl.BlockSpec((BN, Fin), lambda i: (i, 0)),
            pl.BlockSpec((Fin, F), lambda i: (0, 0)),
            pl.BlockSpec((NC, BN), lambda i: (0, i)),
        ],
        out_specs=[
            pl.BlockSpec((BN, F), lambda i: (i, 0)),
            pl.BlockSpec((BN, 1), lambda i: (i, 0)),
        ],
        out_shape=[
            jax.ShapeDtypeStruct((NPAD, F), jnp.float32),
            jax.ShapeDtypeStruct((NPAD, 1), jnp.float32),
        ],
    )(x, W1, deg2)


def _tc_mid_body(agg_ref, hprev_ref, dinv_ref, w_ref, b_ref, hp_ref):
    dv = dinv_ref[...]
    a = agg_ref[0] + agg_ref[1] + hprev_ref[...]
    xl = jnp.maximum(dv * a + b_ref[...], 0.0)
    h = jnp.dot(xl, w_ref[...], preferred_element_type=jnp.float32)
    hp_ref[...] = h * dv


def _tc_mid_call(aggp, hprev, dinv, W, b, Fp, F):
    return pl.pallas_call(
        _tc_mid_body,
        grid=(GRID,),
        in_specs=[
            pl.BlockSpec((NC, BN, Fp), lambda i: (0, i, 0)),
            pl.BlockSpec((BN, Fp), lambda i: (i, 0)),
            pl.BlockSpec((BN, 1), lambda i: (i, 0)),
            pl.BlockSpec((Fp, F), lambda i: (0, 0)),
            pl.BlockSpec((1, Fp), lambda i: (0, 0)),
        ],
        out_specs=pl.BlockSpec((BN, F), lambda i: (i, 0)),
        out_shape=jax.ShapeDtypeStruct((NPAD, F), jnp.float32),
    )(aggp, hprev, dinv, W, b)


def _tc_fin_body(agg_ref, h4_ref, dinv_ref, b_ref, out_ref):
    dv = dinv_ref[...]
    a = agg_ref[0] + agg_ref[1] + h4_ref[...]
    o = dv * a + b_ref[...]
    v = o[:, 0:3]
    m = jnp.max(v, axis=1, keepdims=True)
    s = jnp.sum(jnp.exp(v - m), axis=1, keepdims=True)
    out_ref[...] = v - m - jnp.log(s)


def _tc_fin_call(aggp, h4, dinv, b4p, Fp):
    return pl.pallas_call(
        _tc_fin_body,
        grid=(GRID,),
        in_specs=[
            pl.BlockSpec((NC, BN, Fp), lambda i: (0, i, 0)),
            pl.BlockSpec((BN, Fp), lambda i: (i, 0)),
            pl.BlockSpec((BN, 1), lambda i: (i, 0)),
            pl.BlockSpec((1, Fp), lambda i: (0, 0)),
        ],
        out_specs=pl.BlockSpec((BN, 3), lambda i: (i, 0)),
        out_shape=jax.ShapeDtypeStruct((NPAD, 3), jnp.float32),
    )(aggp, h4, dinv, b4p)


def kernel(x, edge_index, W1, b1, W2, b2, W3, b3, W4, b4):
    f32 = jnp.float32
    e4 = edge_index.reshape(2, NW, CPT, B)
    xp = jnp.pad(x, ((0, NPAD - N), (0, 0)))

    W4p = jnp.zeros((64, 16), f32).at[:, 0:3].set(W4)
    b4p = jnp.zeros((1, 16), f32).at[0, 0:3].set(b4)

    zeros1 = jnp.zeros((NPAD,), f32)
    zf = {F: jnp.zeros((NPAD, F), f32) for F in (16, 32, 64)}

    deg2 = _deg_call(e4, zeros1)

    h1p, dinv = _tc1_call(xp, W1, deg2, 128, 16)
    a1 = _agg_call(h1p, e4, zf[16], 16)
    h2p = _tc_mid_call(a1, h1p, dinv, W2, b1.reshape(1, 16), 16, 32)
    a2 = _agg_call(h2p, e4, zf[32], 32)
    h3p = _tc_mid_call(a2, h2p, dinv, W3, b2.reshape(1, 32), 32, 64)
    a3 = _agg_call(h3p, e4, zf[64], 64)
    h4p = _tc_mid_call(a3, h3p, dinv, W4p, b3.reshape(1, 64), 64, 16)
    a4 = _agg_call(h4p, e4, zf[16], 16)
    out = _tc_fin_call(a4, h4p, dinv, b4p, 16)

    return out[:N]
